# baseline (speedup 1.0000x reference)
import functools

import jax
import jax.numpy as jnp
from jax import lax
from jax.experimental import pallas as pl
from jax.experimental.pallas import tpu as pltpu
from jax.experimental.pallas import tpu_sc as plsc

N = 10000
NPAD = 10112
E = 320000
B = 128
NTILES = 32
ROWS = 2560
RPT = ROWS // NTILES
ERT = ROWS // 16
NBUF = 4
NRND = ERT // NBUF
EPAD = ROWS * B
NROW_T = NPAD // 16
ZR = 320

GB = 10
RB = N // GB

@functools.lru_cache(maxsize=1)
def _mesh():
    return plsc.VectorSubcoreMesh(
        core_axis_name="c", subcore_axis_name="s", num_cores=2,
        num_subcores=16)

_f32 = jnp.float32


def _lr(x):
    return jnp.where(x >= 0, x, 0.01 * x)


def _rb(d):
    return pl.BlockSpec((RB, d), lambda i: (i, 0))


def _full(shape):
    return pl.BlockSpec(shape, lambda i: (0,) * len(shape))


def _deg_call(dst_r):
    def body(dst_hbm, z_hbm, ones_hbm, d0, d1, acc, ones, dstbuf, *dsem):
        cid = lax.axis_index("c")
        sid = lax.axis_index("s")
        wid = cid * 16 + sid
        pltpu.sync_copy(dst_hbm.at[pl.ds(wid * RPT, RPT), :], dstbuf)
        pltpu.sync_copy(ones_hbm, ones)
        pltpu.sync_copy(z_hbm, acc.at[pl.ds(sid * NROW_T, NROW_T), :])
        plsc.subcore_barrier()

        @pl.loop(0, RPT // NBUF)
        def _edges(r):
            sds = [
                pltpu.async_copy(ones, acc.at[dstbuf.at[r * NBUF + b]],
                                 dsem[b], add=True)
                for b in range(NBUF)
            ]
            for d in sds:
                d.wait()

        plsc.subcore_barrier()

        @pl.when(cid == 0)
        def _out0():
            pltpu.sync_copy(acc.at[pl.ds(sid * NROW_T, NROW_T), :],
                            d0.at[pl.ds(sid * NROW_T, NROW_T), :])

        @pl.when(cid == 1)
        def _out1():
            pltpu.sync_copy(acc.at[pl.ds(sid * NROW_T, NROW_T), :],
                            d1.at[pl.ds(sid * NROW_T, NROW_T), :])

    fn = pl.kernel(
        body,
        out_type=(jax.ShapeDtypeStruct((NPAD, 16), _f32),
                  jax.ShapeDtypeStruct((NPAD, 16), _f32)),
        mesh=_mesh(),
        compiler_params=pltpu.CompilerParams(
            use_tc_tiling_on_sc=False, needs_layout_passes=False),
        scratch_types=[
            pltpu.VMEM_SHARED((NPAD, 16), _f32),
            pltpu.VMEM((B, 16), _f32),
            pltpu.VMEM((RPT, B), jnp.int32),
        ] + [pltpu.SemaphoreType.DMA] * NBUF,
    )
    d0, d1 = fn(dst_r, jnp.zeros((NROW_T, 16), _f32),
                jnp.ones((B, 16), _f32))
    return d0[:N], d1[:N]


def _scatter_call(u_list, src_r, dst_r, wc):
    C = len(u_list)
    half = C // 2
    bf = u_list[0].dtype == jnp.bfloat16

    def body(*refs):
        us = refs[:C]
        src_hbm, dst_hbm, z_hbm = refs[C], refs[C + 1], refs[C + 2]
        ss = refs[C + 3:2 * C + 3]
        rest = refs[2 * C + 3:]
        acc, srcbuf, dstbuf = rest[0], rest[1], rest[2]
        rows = rest[3:3 + NBUF]
        if bf:
            rowsf = rest[3 + NBUF:3 + 2 * NBUF]
            sems = rest[3 + 2 * NBUF:]
        else:
            rowsf = rows
            sems = rest[3 + NBUF:]
        gsem = sems[:NBUF]
        ssem = sems[NBUF:2 * NBUF]

        def _upconvert(b):
            @pl.loop(0, B)
            def _row(i):
                for j in range(wc // 32):
                    x = rows[b][i, pl.ds(32 * j, 32)]
                    w = plsc.bitcast(x, jnp.int32)
                    lo = plsc.bitcast(w << 16, _f32)
                    hi = plsc.bitcast(
                        w & jnp.int32(-65536), _f32)
                    rowsf[b][i, pl.ds(32 * j, 16)] = lo
                    rowsf[b][i, pl.ds(32 * j + 16, 16)] = hi
        cid = lax.axis_index("c")
        sid = lax.axis_index("s")
        pltpu.sync_copy(src_hbm.at[pl.ds(sid * ERT, ERT), :], srcbuf)
        pltpu.sync_copy(dst_hbm.at[pl.ds(sid * ERT, ERT), :], dstbuf)
        for k in range(half):
            pltpu.sync_copy(z_hbm, acc.at[pl.ds(sid * NROW_T, NROW_T), :])
            plsc.subcore_barrier()

            for core in range(2):
                @pl.when(cid == core)
                def _work(chunk=core * half + k):
                    @pl.loop(0, NRND)
                    def _round(r):
                        gds = [
                            pltpu.async_copy(
                                us[chunk].at[srcbuf.at[r * NBUF + b]],
                                rows[b], gsem[b])
                            for b in range(NBUF)
                        ]
                        sds = []
                        for b in range(NBUF):
                            gds[b].wait()
                            if bf:
                                _upconvert(b)
                            sds.append(pltpu.async_copy(
                                rowsf[b], acc.at[dstbuf.at[r * NBUF + b]],
                                ssem[b], add=True))
                        for d in sds:
                            d.wait()

            plsc.subcore_barrier()

            for core in range(2):
                @pl.when(cid == core)
                def _out(chunk=core * half + k):
                    pltpu.sync_copy(
                        acc.at[pl.ds(sid * NROW_T, NROW_T), :],
                        ss[chunk].at[pl.ds(sid * NROW_T, NROW_T), :])

            plsc.subcore_barrier()

    fn = pl.kernel(
        body,
        out_type=tuple(jax.ShapeDtypeStruct((NPAD, wc), _f32)
                       for _ in range(C)),
        mesh=_mesh(),
        compiler_params=pltpu.CompilerParams(
            use_tc_tiling_on_sc=False, needs_layout_passes=False),
        scratch_types=(
            [pltpu.VMEM_SHARED((NPAD, wc), _f32),
             pltpu.VMEM((ERT, B), jnp.int32),
             pltpu.VMEM((ERT, B), jnp.int32)]
            + [pltpu.VMEM((B, wc), jnp.bfloat16 if bf else _f32)] * NBUF
            + ([pltpu.VMEM((B, wc), _f32)] * NBUF if bf else [])
            + [pltpu.SemaphoreType.DMA] * (2 * NBUF)
        ),
    )
    z = jnp.zeros((NROW_T, wc), _f32)
    return [o[:N] for o in fn(*u_list, src_r, dst_r, z)]


def _bfperm(u):
    r, w = u.shape
    return (u.reshape(r, w // 32, 2, 16).swapaxes(2, 3).reshape(r, w)
            .astype(jnp.bfloat16))


def _k0_call(x, d0, d1):
    def body(x_ref, d0_ref, d1_ref, dinv_ref, u1a_ref, u1b_ref,
             b1a_ref, b1b_ref):
        deg = d0_ref[:, 0:1] + d1_ref[:, 0:1] + 1.0
        dinv = lax.rsqrt(deg)
        dinv_ref[...] = dinv
        u = x_ref[...] * dinv
        u1a_ref[...] = u[:, :64]
        u1b_ref[...] = u[:, 64:]
        b1a_ref[...] = _bfperm(u[:, :64])
        b1b_ref[...] = _bfperm(u[:, 64:])

    return pl.pallas_call(
        body,
        grid=(GB,),
        out_shape=(jax.ShapeDtypeStruct((N, 1), _f32),
                   jax.ShapeDtypeStruct((N, 64), _f32),
                   jax.ShapeDtypeStruct((N, 64), _f32),
                   jax.ShapeDtypeStruct((N, 64), jnp.bfloat16),
                   jax.ShapeDtypeStruct((N, 64), jnp.bfloat16)),
        in_specs=[_rb(128), _rb(16), _rb(16)],
        out_specs=(_rb(1), _rb(64), _rb(64), _rb(64), _rb(64)),
    )(x, d0, d1)


def _agg_stats_call(s_list, u_list, dinv, W=None):
    C = len(s_list)
    wc = s_list[0].shape[1]
    w = C * wc
    wout = W.shape[1] if W is not None else w

    def body(*refs):
        srefs = refs[:C]
        urefs = refs[C:2 * C]
        dref = refs[2 * C]
        nref = 2 * C + 1
        wref = refs[nref] if W is not None else None
        yref, stref = refs[-2], refs[-1]
        dinv_b = dref[...]
        parts = [dinv_b * (srefs[c][...] + urefs[c][...]) for c in range(C)]
        y = jnp.concatenate(parts, axis=1) if C > 1 else parts[0]
        if W is not None:
            y = jnp.dot(y, wref[...], preferred_element_type=_f32)
        yref[...] = y
        i = pl.program_id(0)

        @pl.when(i == 0)
        def _init():
            stref[...] = jnp.zeros_like(stref)

        stref[0:1, :] += jnp.sum(y, axis=0, keepdims=True)
        stref[1:2, :] += jnp.sum(y * y, axis=0, keepdims=True)

    in_specs = [_rb(wc)] * C + [_rb(wc)] * C + [_rb(1)]
    args = list(s_list) + list(u_list) + [dinv]
    if W is not None:
        in_specs.append(_full(W.shape))
        args.append(W)
    return pl.pallas_call(
        body,
        grid=(GB,),
        out_shape=(jax.ShapeDtypeStruct((N, wout), _f32),
                   jax.ShapeDtypeStruct((2, wout), _f32)),
        in_specs=in_specs,
        out_specs=(_rb(wout), _full((2, wout))),
    )(*args)


def _bn_mm_call(y, st, g, bt, res, Wn, dinv, Cn, want_t):
    w = y.shape[1]
    wn = Wn.shape[1]
    wcn = wn // Cn
    bf = wcn >= 32

    def body(*refs):
        it = iter(refs)
        yref = next(it)
        stref = next(it)
        gref = next(it)
        btref = next(it)
        rref = next(it) if res is not None else None
        wref = next(it)
        dref = next(it)
        outs = list(it)
        m = stref[0:1, :] * (1.0 / N)
        var = jnp.maximum(stref[1:2, :] * (1.0 / N) - m * m, 0.0)
        zh = (yref[...] - m) * lax.rsqrt(var + 1e-5) * gref[...] + btref[...]
        if res is not None:
            zh = zh + rref[...]
        t = _lr(zh)
        oi = 0
        if want_t:
            outs[oi][...] = t
            oi += 1
        u = dref[...] * jnp.dot(t, wref[...], preferred_element_type=_f32)
        for c in range(Cn):
            outs[oi + c][...] = u[:, c * wcn:(c + 1) * wcn]
        if bf:
            for c in range(Cn):
                outs[oi + Cn + c][...] = _bfperm(u[:, c * wcn:(c + 1) * wcn])

    in_specs = [_rb(w), _full((2, w)), _full((1, w)), _full((1, w))]
    args = [y, st, g, bt]
    if res is not None:
        in_specs.append(_rb(w))
        args.append(res)
    in_specs += [_full(Wn.shape), _rb(1)]
    args += [Wn, dinv]
    out_shape = []
    out_specs = []
    if want_t:
        out_shape.append(jax.ShapeDtypeStruct((N, w), _f32))
        out_specs.append(_rb(w))
    out_shape += [jax.ShapeDtypeStruct((N, wcn), _f32)] * Cn
    out_specs += [_rb(wcn)] * Cn
    if bf:
        out_shape += [jax.ShapeDtypeStruct((N, wcn), jnp.bfloat16)] * Cn
        out_specs += [_rb(wcn)] * Cn
    outs = pl.pallas_call(
        body,
        grid=(GB,),
        out_shape=tuple(out_shape),
        in_specs=in_specs,
        out_specs=tuple(out_specs),
    )(*args)
    outs = list(outs)
    t = outs[0] if want_t else None
    rest = outs[1:] if want_t else outs
    uf = rest[:Cn]
    ub = rest[Cn:] if bf else uf
    return t, uf, ub


def _final_call(y, st, g, bt, gf, wg1, bg1, wg2, bg2, wfc, bfc):
    def body(yref, stref, gref, btref, gfref, wg1ref, bg1ref, wg2ref,
             bg2ref, wfcref, bfcref, outref):
        m = stref[0:1, :] * (1.0 / N)
        var = jnp.maximum(stref[1:2, :] * (1.0 / N) - m * m, 0.0)
        zh = (yref[...] - m) * lax.rsqrt(var + 1e-5) * gref[...] + btref[...]
        t = _lr(zh)
        gg = _lr(jnp.dot(gfref[...], wg1ref[...],
                         preferred_element_type=_f32) + bg1ref[...])
        gg = jnp.dot(gg, wg2ref[...], preferred_element_type=_f32) + bg2ref[...]
        wfc = wfcref[...]
        logits = (jnp.dot(t, wfc[:32, :], preferred_element_type=_f32)
                  + gg * wfc[32:33, :] + bfcref[...])
        outref[...] = jax.nn.sigmoid(logits)

    return pl.pallas_call(
        body,
        grid=(GB,),
        out_shape=jax.ShapeDtypeStruct((N, 1), _f32),
        in_specs=[_rb(32), _full((2, 32)), _full((1, 32)), _full((1, 32)),
                  _rb(1), _full((1, 64)), _full((1, 64)), _full((64, 1)),
                  _full((1, 1)), _full((33, 1)), _full((1, 1))],
        out_specs=_rb(1),
    )(y, st, g, bt, gf, wg1, bg1, wg2, bg2, wfc, bfc)


def kernel(x, edge_index, global_features, params):
    p = params
    src = edge_index[0].astype(jnp.int32)
    dst = edge_index[1].astype(jnp.int32)
    pad = EPAD - E
    src_r = jnp.concatenate(
        [src, jnp.zeros((pad,), jnp.int32)]).reshape(ROWS, B)
    dst_r = jnp.concatenate(
        [dst, jnp.full((pad,), N, jnp.int32)]).reshape(ROWS, B)

    def r2(v):
        return v.reshape(1, -1)

    d0, d1 = _deg_call(dst_r)
    dinv, u1a, u1b, b1a, b1b = _k0_call(x, d0, d1)

    s1 = _scatter_call([b1a, b1b], src_r, dst_r, 64)
    z1, st1 = _agg_stats_call(s1, [u1a, u1b], dinv, W=p["W1"])
    t1, u2, u2b = _bn_mm_call(z1, st1, r2(p["g1"]), r2(p["bt1"]), None,
                              p["W2"], dinv, Cn=8, want_t=True)

    s2 = _scatter_call(u2b, src_r, dst_r, 64)
    y2, st2 = _agg_stats_call(s2, u2, dinv)
    _, u3, u3b = _bn_mm_call(y2, st2, r2(p["g2"]), r2(p["bt2"]), t1,
                             p["W3"], dinv, Cn=4, want_t=False)

    s3 = _scatter_call(u3b, src_r, dst_r, 64)
    y3, st3 = _agg_stats_call(s3, u3, dinv)
    _, u4, u4b = _bn_mm_call(y3, st3, r2(p["g3"]), r2(p["bt3"]), None,
                             p["W4"], dinv, Cn=2, want_t=False)

    s4 = _scatter_call(u4b, src_r, dst_r, 64)
    y4, st4 = _agg_stats_call(s4, u4, dinv)
    _, u5, u5b = _bn_mm_call(y4, st4, r2(p["g4"]), r2(p["bt4"]), None,
                             p["W5"], dinv, Cn=2, want_t=False)

    s5 = _scatter_call(u5b, src_r, dst_r, 32)
    y5, st5 = _agg_stats_call(s5, u5, dinv)
    _, u6, _u6b = _bn_mm_call(y5, st5, r2(p["g5"]), r2(p["bt5"]), None,
                              p["W6"], dinv, Cn=2, want_t=False)

    s6 = _scatter_call(u6, src_r, dst_r, 16)
    y6, st6 = _agg_stats_call(s6, u6, dinv)
    return _final_call(
        y6, st6, r2(p["g6"]), r2(p["bt6"]), global_features,
        p["Wg1"], r2(p["bg1"]), p["Wg2"], r2(p["bg2"]),
        p["Wfc"], r2(p["bfc"]))


# speedup vs baseline: 1.4721x; 1.4721x over previous
import functools

import jax
import jax.numpy as jnp
from jax import lax
from jax.experimental import pallas as pl
from jax.experimental.pallas import tpu as pltpu
from jax.experimental.pallas import tpu_sc as plsc

N = 10000
NPAD = 10112
E = 320000
B = 128
NTILES = 32
ROWS = 2560
RPT = ROWS // NTILES
ERT = ROWS // 16
NBUF = 5
NRND = ERT // NBUF
EPAD = ROWS * B
NROW_T = NPAD // 16
ZR = 320

GB = 5
RB = N // GB

@functools.lru_cache(maxsize=1)
def _mesh():
    return plsc.VectorSubcoreMesh(
        core_axis_name="c", subcore_axis_name="s", num_cores=2,
        num_subcores=16)

_f32 = jnp.float32


def _lr(x):
    return jnp.where(x >= 0, x, 0.01 * x)


def _rb(d):
    return pl.BlockSpec((RB, d), lambda i: (i, 0))


def _full(shape):
    return pl.BlockSpec(shape, lambda i: (0,) * len(shape))


def _deg_call(dst_r):
    def body(dst_hbm, z_hbm, ones_hbm, d0, d1, acc, ones, dstbuf, *dsem):
        cid = lax.axis_index("c")
        sid = lax.axis_index("s")
        wid = cid * 16 + sid
        pltpu.sync_copy(dst_hbm.at[pl.ds(wid * RPT, RPT), :], dstbuf)
        pltpu.sync_copy(ones_hbm, ones)
        pltpu.sync_copy(z_hbm, acc.at[pl.ds(sid * NROW_T, NROW_T), :])
        plsc.subcore_barrier()

        @pl.loop(0, RPT // NBUF)
        def _edges(r):
            sds = [
                pltpu.async_copy(ones, acc.at[dstbuf.at[r * NBUF + b]],
                                 dsem[b], add=True)
                for b in range(NBUF)
            ]
            for d in sds:
                d.wait()

        plsc.subcore_barrier()

        @pl.when(cid == 0)
        def _out0():
            pltpu.sync_copy(acc.at[pl.ds(sid * NROW_T, NROW_T), :],
                            d0.at[pl.ds(sid * NROW_T, NROW_T), :])

        @pl.when(cid == 1)
        def _out1():
            pltpu.sync_copy(acc.at[pl.ds(sid * NROW_T, NROW_T), :],
                            d1.at[pl.ds(sid * NROW_T, NROW_T), :])

    fn = pl.kernel(
        body,
        out_type=(jax.ShapeDtypeStruct((NPAD, 16), _f32),
                  jax.ShapeDtypeStruct((NPAD, 16), _f32)),
        mesh=_mesh(),
        compiler_params=pltpu.CompilerParams(use_tc_tiling_on_sc=False),
        scratch_types=[
            pltpu.VMEM_SHARED((NPAD, 16), _f32),
            pltpu.VMEM((B, 16), _f32),
            pltpu.VMEM((RPT, B), jnp.int32),
        ] + [pltpu.SemaphoreType.DMA] * NBUF,
    )
    d0, d1 = fn(dst_r, jnp.zeros((NROW_T, 16), _f32),
                jnp.ones((B, 16), _f32))
    return d0[:N], d1[:N]


def _scatter_call(u_list, src_r, dst_r, wc):
    C = len(u_list)
    half = C // 2

    def body(*refs):
        us = refs[:C]
        src_hbm, dst_hbm, z_hbm = refs[C], refs[C + 1], refs[C + 2]
        ss = refs[C + 3:2 * C + 3]
        rest = refs[2 * C + 3:]
        acc, srcbuf, dstbuf = rest[0], rest[1], rest[2]
        rows = rest[3:3 + NBUF]
        gsem = rest[3 + NBUF:3 + 2 * NBUF]
        ssem = rest[3 + 2 * NBUF:3 + 3 * NBUF]
        cid = lax.axis_index("c")
        sid = lax.axis_index("s")
        pltpu.sync_copy(src_hbm.at[pl.ds(sid * ERT, ERT), :], srcbuf)
        pltpu.sync_copy(dst_hbm.at[pl.ds(sid * ERT, ERT), :], dstbuf)
        for k in range(half):
            pltpu.sync_copy(z_hbm, acc.at[pl.ds(sid * NROW_T, NROW_T), :])
            plsc.subcore_barrier()

            for core in range(2):
                @pl.when(cid == core)
                def _work(chunk=core * half + k):
                    @pl.loop(0, NRND)
                    def _round(r):
                        gds = [
                            pltpu.async_copy(
                                us[chunk].at[srcbuf.at[r * NBUF + b]],
                                rows[b], gsem[b])
                            for b in range(NBUF)
                        ]
                        sds = []
                        for b in range(NBUF):
                            gds[b].wait()
                            sds.append(pltpu.async_copy(
                                rows[b], acc.at[dstbuf.at[r * NBUF + b]],
                                ssem[b], add=True))
                        for d in sds:
                            d.wait()

            plsc.subcore_barrier()

            for core in range(2):
                @pl.when(cid == core)
                def _out(chunk=core * half + k):
                    pltpu.sync_copy(
                        acc.at[pl.ds(sid * NROW_T, NROW_T), :],
                        ss[chunk].at[pl.ds(sid * NROW_T, NROW_T), :])

            plsc.subcore_barrier()

    fn = pl.kernel(
        body,
        out_type=tuple(jax.ShapeDtypeStruct((NPAD, wc), _f32)
                       for _ in range(C)),
        mesh=_mesh(),
        compiler_params=pltpu.CompilerParams(use_tc_tiling_on_sc=False),
        scratch_types=(
            [pltpu.VMEM_SHARED((NPAD, wc), _f32),
             pltpu.VMEM((ERT, B), jnp.int32),
             pltpu.VMEM((ERT, B), jnp.int32)]
            + [pltpu.VMEM((B, wc), _f32)] * NBUF
            + [pltpu.SemaphoreType.DMA] * (2 * NBUF)
        ),
    )
    z = jnp.zeros((NROW_T, wc), _f32)
    return [o[:N] for o in fn(*u_list, src_r, dst_r, z)]


def _k0_call(x, d0, d1):
    def body(x_ref, d0_ref, d1_ref, dinv_ref, u1a_ref, u1b_ref):
        deg = d0_ref[:, 0:1] + d1_ref[:, 0:1] + 1.0
        dinv = lax.rsqrt(deg)
        dinv_ref[...] = dinv
        u = x_ref[...] * dinv
        u1a_ref[...] = u[:, :64]
        u1b_ref[...] = u[:, 64:]

    return pl.pallas_call(
        body,
        grid=(GB,),
        out_shape=(jax.ShapeDtypeStruct((N, 1), _f32),
                   jax.ShapeDtypeStruct((N, 64), _f32),
                   jax.ShapeDtypeStruct((N, 64), _f32)),
        in_specs=[_rb(128), _rb(16), _rb(16)],
        out_specs=(_rb(1), _rb(64), _rb(64)),
    )(x, d0, d1)


def _agg_stats_call(s_list, u_list, dinv, W=None):
    C = len(s_list)
    wc = s_list[0].shape[1]
    w = C * wc
    wout = W.shape[1] if W is not None else w

    def body(*refs):
        srefs = refs[:C]
        urefs = refs[C:2 * C]
        dref = refs[2 * C]
        nref = 2 * C + 1
        wref = refs[nref] if W is not None else None
        yref, stref = refs[-2], refs[-1]
        dinv_b = dref[...]
        parts = [dinv_b * (srefs[c][...] + urefs[c][...]) for c in range(C)]
        y = jnp.concatenate(parts, axis=1) if C > 1 else parts[0]
        if W is not None:
            y = jnp.dot(y, wref[...], preferred_element_type=_f32)
        yref[...] = y
        i = pl.program_id(0)

        @pl.when(i == 0)
        def _init():
            stref[...] = jnp.zeros_like(stref)

        stref[0:1, :] += jnp.sum(y, axis=0, keepdims=True)
        stref[1:2, :] += jnp.sum(y * y, axis=0, keepdims=True)

    in_specs = [_rb(wc)] * C + [_rb(wc)] * C + [_rb(1)]
    args = list(s_list) + list(u_list) + [dinv]
    if W is not None:
        in_specs.append(_full(W.shape))
        args.append(W)
    return pl.pallas_call(
        body,
        grid=(GB,),
        out_shape=(jax.ShapeDtypeStruct((N, wout), _f32),
                   jax.ShapeDtypeStruct((2, wout), _f32)),
        in_specs=in_specs,
        out_specs=(_rb(wout), _full((2, wout))),
    )(*args)


def _bn_mm_call(y, st, g, bt, res, Wn, dinv, Cn, want_t):
    w = y.shape[1]
    wn = Wn.shape[1]
    wcn = wn // Cn

    def body(*refs):
        it = iter(refs)
        yref = next(it)
        stref = next(it)
        gref = next(it)
        btref = next(it)
        rref = next(it) if res is not None else None
        wref = next(it)
        dref = next(it)
        outs = list(it)
        m = stref[0:1, :] * (1.0 / N)
        var = jnp.maximum(stref[1:2, :] * (1.0 / N) - m * m, 0.0)
        zh = (yref[...] - m) * lax.rsqrt(var + 1e-5) * gref[...] + btref[...]
        if res is not None:
            zh = zh + rref[...]
        t = _lr(zh)
        oi = 0
        if want_t:
            outs[oi][...] = t
            oi += 1
        u = dref[...] * jnp.dot(t, wref[...], preferred_element_type=_f32)
        for c in range(Cn):
            outs[oi + c][...] = u[:, c * wcn:(c + 1) * wcn]

    in_specs = [_rb(w), _full((2, w)), _full((1, w)), _full((1, w))]
    args = [y, st, g, bt]
    if res is not None:
        in_specs.append(_rb(w))
        args.append(res)
    in_specs += [_full(Wn.shape), _rb(1)]
    args += [Wn, dinv]
    out_shape = []
    out_specs = []
    if want_t:
        out_shape.append(jax.ShapeDtypeStruct((N, w), _f32))
        out_specs.append(_rb(w))
    out_shape += [jax.ShapeDtypeStruct((N, wcn), _f32)] * Cn
    out_specs += [_rb(wcn)] * Cn
    outs = pl.pallas_call(
        body,
        grid=(GB,),
        out_shape=tuple(out_shape),
        in_specs=in_specs,
        out_specs=tuple(out_specs),
    )(*args)
    outs = list(outs)
    if want_t:
        return outs[0], outs[1:]
    return None, outs


def _final_call(y, st, g, bt, gf, wg1, bg1, wg2, bg2, wfc, bfc):
    def body(yref, stref, gref, btref, gfref, wg1ref, bg1ref, wg2ref,
             bg2ref, wfcref, bfcref, outref):
        m = stref[0:1, :] * (1.0 / N)
        var = jnp.maximum(stref[1:2, :] * (1.0 / N) - m * m, 0.0)
        zh = (yref[...] - m) * lax.rsqrt(var + 1e-5) * gref[...] + btref[...]
        t = _lr(zh)
        gg = _lr(jnp.dot(gfref[...], wg1ref[...],
                         preferred_element_type=_f32) + bg1ref[...])
        gg = jnp.dot(gg, wg2ref[...], preferred_element_type=_f32) + bg2ref[...]
        wfc = wfcref[...]
        logits = (jnp.dot(t, wfc[:32, :], preferred_element_type=_f32)
                  + gg * wfc[32:33, :] + bfcref[...])
        outref[...] = jax.nn.sigmoid(logits)

    return pl.pallas_call(
        body,
        grid=(GB,),
        out_shape=jax.ShapeDtypeStruct((N, 1), _f32),
        in_specs=[_rb(32), _full((2, 32)), _full((1, 32)), _full((1, 32)),
                  _rb(1), _full((1, 64)), _full((1, 64)), _full((64, 1)),
                  _full((1, 1)), _full((33, 1)), _full((1, 1))],
        out_specs=_rb(1),
    )(y, st, g, bt, gf, wg1, bg1, wg2, bg2, wfc, bfc)


def kernel(x, edge_index, global_features, params):
    p = params
    src = edge_index[0].astype(jnp.int32)
    dst = edge_index[1].astype(jnp.int32)
    pad = EPAD - E
    src_r = jnp.concatenate(
        [src, jnp.zeros((pad,), jnp.int32)]).reshape(ROWS, B)
    dst_r = jnp.concatenate(
        [dst, jnp.full((pad,), N, jnp.int32)]).reshape(ROWS, B)

    def r2(v):
        return v.reshape(1, -1)

    d0, d1 = _deg_call(dst_r)
    dinv, u1a, u1b = _k0_call(x, d0, d1)

    s1 = _scatter_call([u1a, u1b], src_r, dst_r, 64)
    z1, st1 = _agg_stats_call(s1, [u1a, u1b], dinv, W=p["W1"])
    t1, u2 = _bn_mm_call(z1, st1, r2(p["g1"]), r2(p["bt1"]), None,
                         p["W2"], dinv, Cn=8, want_t=True)

    s2 = _scatter_call(u2, src_r, dst_r, 64)
    y2, st2 = _agg_stats_call(s2, u2, dinv)
    _, u3 = _bn_mm_call(y2, st2, r2(p["g2"]), r2(p["bt2"]), t1,
                        p["W3"], dinv, Cn=4, want_t=False)

    s3 = _scatter_call(u3, src_r, dst_r, 64)
    y3, st3 = _agg_stats_call(s3, u3, dinv)
    _, u4 = _bn_mm_call(y3, st3, r2(p["g3"]), r2(p["bt3"]), None,
                        p["W4"], dinv, Cn=2, want_t=False)

    s4 = _scatter_call(u4, src_r, dst_r, 64)
    y4, st4 = _agg_stats_call(s4, u4, dinv)
    _, u5 = _bn_mm_call(y4, st4, r2(p["g4"]), r2(p["bt4"]), None,
                        p["W5"], dinv, Cn=2, want_t=False)

    s5 = _scatter_call(u5, src_r, dst_r, 32)
    y5, st5 = _agg_stats_call(s5, u5, dinv)
    _, u6 = _bn_mm_call(y5, st5, r2(p["g5"]), r2(p["bt5"]), None,
                        p["W6"], dinv, Cn=2, want_t=False)

    s6 = _scatter_call(u6, src_r, dst_r, 16)
    y6, st6 = _agg_stats_call(s6, u6, dinv)
    return _final_call(
        y6, st6, r2(p["g6"]), r2(p["bt6"]), global_features,
        p["Wg1"], r2(p["bg1"]), p["Wg2"], r2(p["bg2"]),
        p["Wfc"], r2(p["bfc"]))


# speedup vs baseline: 1.4743x; 1.0015x over previous
import functools

import jax
import jax.numpy as jnp
from jax import lax
from jax.experimental import pallas as pl
from jax.experimental.pallas import tpu as pltpu
from jax.experimental.pallas import tpu_sc as plsc

N = 10000
NPAD = 10112
E = 320000
B = 128
NTILES = 32
ROWS = 2560
RPT = ROWS // NTILES
ERT = ROWS // 16
NBUF = 8
HH = 2
NRND = ERT // NBUF
EPAD = ROWS * B
NROW_T = NPAD // 16
ZR = 320

GB = 5
RB = N // GB

@functools.lru_cache(maxsize=1)
def _mesh():
    return plsc.VectorSubcoreMesh(
        core_axis_name="c", subcore_axis_name="s", num_cores=2,
        num_subcores=16)

_f32 = jnp.float32


def _lr(x):
    return jnp.where(x >= 0, x, 0.01 * x)


def _rb(d):
    return pl.BlockSpec((RB, d), lambda i: (i, 0))


def _full(shape):
    return pl.BlockSpec(shape, lambda i: (0,) * len(shape))


def _deg_call(dst_r):
    def body(dst_hbm, z_hbm, ones_hbm, d0, d1, acc, ones, dstbuf, *dsem):
        cid = lax.axis_index("c")
        sid = lax.axis_index("s")
        wid = cid * 16 + sid
        pltpu.sync_copy(dst_hbm.at[pl.ds(wid * RPT, RPT), :], dstbuf)
        pltpu.sync_copy(ones_hbm, ones)
        pltpu.sync_copy(z_hbm, acc.at[pl.ds(sid * NROW_T, NROW_T), :])
        plsc.subcore_barrier()

        @pl.loop(0, RPT // NBUF)
        def _edges(r):
            sds = [
                pltpu.async_copy(ones, acc.at[dstbuf.at[r * NBUF + b]],
                                 dsem[b], add=True)
                for b in range(NBUF)
            ]
            for d in sds:
                d.wait()

        plsc.subcore_barrier()

        @pl.when(cid == 0)
        def _out0():
            pltpu.sync_copy(acc.at[pl.ds(sid * NROW_T, NROW_T), :],
                            d0.at[pl.ds(sid * NROW_T, NROW_T), :])

        @pl.when(cid == 1)
        def _out1():
            pltpu.sync_copy(acc.at[pl.ds(sid * NROW_T, NROW_T), :],
                            d1.at[pl.ds(sid * NROW_T, NROW_T), :])

    fn = pl.kernel(
        body,
        out_type=(jax.ShapeDtypeStruct((NPAD, 16), _f32),
                  jax.ShapeDtypeStruct((NPAD, 16), _f32)),
        mesh=_mesh(),
        compiler_params=pltpu.CompilerParams(use_tc_tiling_on_sc=False),
        scratch_types=[
            pltpu.VMEM_SHARED((NPAD, 16), _f32),
            pltpu.VMEM((B, 16), _f32),
            pltpu.VMEM((RPT, B), jnp.int32),
        ] + [pltpu.SemaphoreType.DMA] * NBUF,
    )
    d0, d1 = fn(dst_r, jnp.zeros((NROW_T, 16), _f32),
                jnp.ones((B, 16), _f32))
    return d0[:N], d1[:N]


def _scatter_call(u_list, src_r, dst_r, wc):
    C = len(u_list)
    half = C // 2

    def body(*refs):
        us = refs[:C]
        src_hbm, dst_hbm, z_hbm = refs[C], refs[C + 1], refs[C + 2]
        ss = refs[C + 3:2 * C + 3]
        rest = refs[2 * C + 3:]
        acc, srcbuf, dstbuf = rest[0], rest[1], rest[2]
        rows = rest[3:3 + NBUF]
        gsem = rest[3 + NBUF:3 + 2 * NBUF]
        ssem = rest[3 + 2 * NBUF:3 + 3 * NBUF]
        cid = lax.axis_index("c")
        sid = lax.axis_index("s")
        for k in range(half):
            pltpu.sync_copy(z_hbm, acc.at[pl.ds(sid * NROW_T, NROW_T), :])
            plsc.subcore_barrier()

            for core in range(2):
                @pl.when(cid == core)
                def _work(chunk=core * half + k):
                    for hh in range(HH):
                        base = sid * ERT + hh * (ERT // HH)
                        pltpu.sync_copy(
                            src_hbm.at[pl.ds(base, ERT // HH), :], srcbuf)
                        pltpu.sync_copy(
                            dst_hbm.at[pl.ds(base, ERT // HH), :], dstbuf)

                        @pl.loop(0, ERT // HH // NBUF)
                        def _round(r):
                            gds = [
                                pltpu.async_copy(
                                    us[chunk].at[srcbuf.at[r * NBUF + b]],
                                    rows[b], gsem[b])
                                for b in range(NBUF)
                            ]
                            sds = []
                            for b in range(NBUF):
                                gds[b].wait()
                                sds.append(pltpu.async_copy(
                                    rows[b],
                                    acc.at[dstbuf.at[r * NBUF + b]],
                                    ssem[b], add=True))
                            for d in sds:
                                d.wait()

            plsc.subcore_barrier()

            for core in range(2):
                @pl.when(cid == core)
                def _out(chunk=core * half + k):
                    pltpu.sync_copy(
                        acc.at[pl.ds(sid * NROW_T, NROW_T), :],
                        ss[chunk].at[pl.ds(sid * NROW_T, NROW_T), :])

            plsc.subcore_barrier()

    fn = pl.kernel(
        body,
        out_type=tuple(jax.ShapeDtypeStruct((NPAD, wc), _f32)
                       for _ in range(C)),
        mesh=_mesh(),
        compiler_params=pltpu.CompilerParams(use_tc_tiling_on_sc=False),
        scratch_types=(
            [pltpu.VMEM_SHARED((NPAD, wc), _f32),
             pltpu.VMEM((ERT // HH, B), jnp.int32),
             pltpu.VMEM((ERT // HH, B), jnp.int32)]
            + [pltpu.VMEM((B, wc), _f32)] * NBUF
            + [pltpu.SemaphoreType.DMA] * (2 * NBUF)
        ),
    )
    z = jnp.zeros((NROW_T, wc), _f32)
    return [o[:N] for o in fn(*u_list, src_r, dst_r, z)]


def _k0_call(x, d0, d1):
    def body(x_ref, d0_ref, d1_ref, dinv_ref, u1a_ref, u1b_ref):
        deg = d0_ref[:, 0:1] + d1_ref[:, 0:1] + 1.0
        dinv = lax.rsqrt(deg)
        dinv_ref[...] = dinv
        u = x_ref[...] * dinv
        u1a_ref[...] = u[:, :64]
        u1b_ref[...] = u[:, 64:]

    return pl.pallas_call(
        body,
        grid=(GB,),
        out_shape=(jax.ShapeDtypeStruct((N, 1), _f32),
                   jax.ShapeDtypeStruct((N, 64), _f32),
                   jax.ShapeDtypeStruct((N, 64), _f32)),
        in_specs=[_rb(128), _rb(16), _rb(16)],
        out_specs=(_rb(1), _rb(64), _rb(64)),
    )(x, d0, d1)


def _agg_stats_call(s_list, u_list, dinv, W=None):
    C = len(s_list)
    wc = s_list[0].shape[1]
    w = C * wc
    wout = W.shape[1] if W is not None else w

    def body(*refs):
        srefs = refs[:C]
        urefs = refs[C:2 * C]
        dref = refs[2 * C]
        nref = 2 * C + 1
        wref = refs[nref] if W is not None else None
        yref, stref = refs[-2], refs[-1]
        dinv_b = dref[...]
        parts = [dinv_b * (srefs[c][...] + urefs[c][...]) for c in range(C)]
        y = jnp.concatenate(parts, axis=1) if C > 1 else parts[0]
        if W is not None:
            y = jnp.dot(y, wref[...], preferred_element_type=_f32)
        yref[...] = y
        i = pl.program_id(0)

        @pl.when(i == 0)
        def _init():
            stref[...] = jnp.zeros_like(stref)

        stref[0:1, :] += jnp.sum(y, axis=0, keepdims=True)
        stref[1:2, :] += jnp.sum(y * y, axis=0, keepdims=True)

    in_specs = [_rb(wc)] * C + [_rb(wc)] * C + [_rb(1)]
    args = list(s_list) + list(u_list) + [dinv]
    if W is not None:
        in_specs.append(_full(W.shape))
        args.append(W)
    return pl.pallas_call(
        body,
        grid=(GB,),
        out_shape=(jax.ShapeDtypeStruct((N, wout), _f32),
                   jax.ShapeDtypeStruct((2, wout), _f32)),
        in_specs=in_specs,
        out_specs=(_rb(wout), _full((2, wout))),
    )(*args)


def _bn_mm_call(y, st, g, bt, res, Wn, dinv, Cn, want_t):
    w = y.shape[1]
    wn = Wn.shape[1]
    wcn = wn // Cn

    def body(*refs):
        it = iter(refs)
        yref = next(it)
        stref = next(it)
        gref = next(it)
        btref = next(it)
        rref = next(it) if res is not None else None
        wref = next(it)
        dref = next(it)
        outs = list(it)
        m = stref[0:1, :] * (1.0 / N)
        var = jnp.maximum(stref[1:2, :] * (1.0 / N) - m * m, 0.0)
        zh = (yref[...] - m) * lax.rsqrt(var + 1e-5) * gref[...] + btref[...]
        if res is not None:
            zh = zh + rref[...]
        t = _lr(zh)
        oi = 0
        if want_t:
            outs[oi][...] = t
            oi += 1
        u = dref[...] * jnp.dot(t, wref[...], preferred_element_type=_f32)
        for c in range(Cn):
            outs[oi + c][...] = u[:, c * wcn:(c + 1) * wcn]

    in_specs = [_rb(w), _full((2, w)), _full((1, w)), _full((1, w))]
    args = [y, st, g, bt]
    if res is not None:
        in_specs.append(_rb(w))
        args.append(res)
    in_specs += [_full(Wn.shape), _rb(1)]
    args += [Wn, dinv]
    out_shape = []
    out_specs = []
    if want_t:
        out_shape.append(jax.ShapeDtypeStruct((N, w), _f32))
        out_specs.append(_rb(w))
    out_shape += [jax.ShapeDtypeStruct((N, wcn), _f32)] * Cn
    out_specs += [_rb(wcn)] * Cn
    outs = pl.pallas_call(
        body,
        grid=(GB,),
        out_shape=tuple(out_shape),
        in_specs=in_specs,
        out_specs=tuple(out_specs),
    )(*args)
    outs = list(outs)
    if want_t:
        return outs[0], outs[1:]
    return None, outs


def _final_call(y, st, g, bt, gf, wg1, bg1, wg2, bg2, wfc, bfc):
    def body(yref, stref, gref, btref, gfref, wg1ref, bg1ref, wg2ref,
             bg2ref, wfcref, bfcref, outref):
        m = stref[0:1, :] * (1.0 / N)
        var = jnp.maximum(stref[1:2, :] * (1.0 / N) - m * m, 0.0)
        zh = (yref[...] - m) * lax.rsqrt(var + 1e-5) * gref[...] + btref[...]
        t = _lr(zh)
        gg = _lr(jnp.dot(gfref[...], wg1ref[...],
                         preferred_element_type=_f32) + bg1ref[...])
        gg = jnp.dot(gg, wg2ref[...], preferred_element_type=_f32) + bg2ref[...]
        wfc = wfcref[...]
        logits = (jnp.dot(t, wfc[:32, :], preferred_element_type=_f32)
                  + gg * wfc[32:33, :] + bfcref[...])
        outref[...] = jax.nn.sigmoid(logits)

    return pl.pallas_call(
        body,
        grid=(GB,),
        out_shape=jax.ShapeDtypeStruct((N, 1), _f32),
        in_specs=[_rb(32), _full((2, 32)), _full((1, 32)), _full((1, 32)),
                  _rb(1), _full((1, 64)), _full((1, 64)), _full((64, 1)),
                  _full((1, 1)), _full((33, 1)), _full((1, 1))],
        out_specs=_rb(1),
    )(y, st, g, bt, gf, wg1, bg1, wg2, bg2, wfc, bfc)


def kernel(x, edge_index, global_features, params):
    p = params
    src = edge_index[0].astype(jnp.int32)
    dst = edge_index[1].astype(jnp.int32)
    pad = EPAD - E
    src_r = jnp.concatenate(
        [src, jnp.zeros((pad,), jnp.int32)]).reshape(ROWS, B)
    dst_r = jnp.concatenate(
        [dst, jnp.full((pad,), N, jnp.int32)]).reshape(ROWS, B)

    def r2(v):
        return v.reshape(1, -1)

    d0, d1 = _deg_call(dst_r)
    dinv, u1a, u1b = _k0_call(x, d0, d1)

    s1 = _scatter_call([u1a, u1b], src_r, dst_r, 64)
    z1, st1 = _agg_stats_call(s1, [u1a, u1b], dinv, W=p["W1"])
    t1, u2 = _bn_mm_call(z1, st1, r2(p["g1"]), r2(p["bt1"]), None,
                         p["W2"], dinv, Cn=8, want_t=True)

    s2 = _scatter_call(u2, src_r, dst_r, 64)
    y2, st2 = _agg_stats_call(s2, u2, dinv)
    _, u3 = _bn_mm_call(y2, st2, r2(p["g2"]), r2(p["bt2"]), t1,
                        p["W3"], dinv, Cn=4, want_t=False)

    s3 = _scatter_call(u3, src_r, dst_r, 64)
    y3, st3 = _agg_stats_call(s3, u3, dinv)
    _, u4 = _bn_mm_call(y3, st3, r2(p["g3"]), r2(p["bt3"]), None,
                        p["W4"], dinv, Cn=2, want_t=False)

    s4 = _scatter_call(u4, src_r, dst_r, 64)
    y4, st4 = _agg_stats_call(s4, u4, dinv)
    _, u5 = _bn_mm_call(y4, st4, r2(p["g4"]), r2(p["bt4"]), None,
                        p["W5"], dinv, Cn=2, want_t=False)

    s5 = _scatter_call(u5, src_r, dst_r, 32)
    y5, st5 = _agg_stats_call(s5, u5, dinv)
    _, u6 = _bn_mm_call(y5, st5, r2(p["g5"]), r2(p["bt5"]), None,
                        p["W6"], dinv, Cn=2, want_t=False)

    s6 = _scatter_call(u6, src_r, dst_r, 16)
    y6, st6 = _agg_stats_call(s6, u6, dinv)
    return _final_call(
        y6, st6, r2(p["g6"]), r2(p["bt6"]), global_features,
        p["Wg1"], r2(p["bg1"]), p["Wg2"], r2(p["bg2"]),
        p["Wfc"], r2(p["bfc"]))
